# Initial kernel scaffold; baseline (speedup 1.0000x reference)
import jax
import jax.numpy as jnp
from jax.experimental import pallas as pl


def kernel(x, table):
    raise NotImplementedError("write your pallas kernel here")


# speedup vs baseline: 1.1138x; 1.1138x over previous
import functools

import jax
import jax.numpy as jnp
from jax import lax
from jax.experimental import pallas as pl
from jax.experimental.pallas import tpu as pltpu
from jax.experimental.pallas import tpu_sc as plsc

_D = 32
_NC, _NS = 2, 16
_NW = _NC * _NS
_CH = 1280
_NBUF = 2


@jax.jit
def _embed_gather(idx, table):
    b = idx.shape[0]
    bpw = b // _NW
    nch = bpw // _CH
    mesh = plsc.VectorSubcoreMesh(core_axis_name="c", subcore_axis_name="s")

    @functools.partial(
        pl.kernel,
        out_type=jax.ShapeDtypeStruct((b, _D), jnp.float32),
        mesh=mesh,
        scratch_types=[
            pltpu.VMEM((bpw,), jnp.int32),
            pltpu.VMEM((_NBUF, _CH, _D), jnp.float32),
            pltpu.SemaphoreType.DMA,
            pltpu.SemaphoreType.DMA,
        ],
        compiler_params=pltpu.CompilerParams(use_tc_tiling_on_sc=False),
    )
    def k(idx_hbm, table_hbm, out_hbm, idx_v, rows_v, sem0, sem1):
        sems = (sem0, sem1)
        wid = lax.axis_index("s") * _NC + lax.axis_index("c")
        base = wid * bpw
        pltpu.sync_copy(idx_hbm.at[pl.ds(base, bpw)], idx_v)

        def start_gather(buf, c):
            pltpu.async_copy(table_hbm.at[idx_v.at[pl.ds(c * _CH, _CH)]],
                             rows_v.at[buf], sems[buf])

        def wait_gather(buf):
            pltpu.make_async_copy(table_hbm.at[idx_v.at[pl.ds(0, _CH)]],
                                  rows_v.at[buf], sems[buf]).wait()

        for buf in range(_NBUF):
            start_gather(buf, buf)

        @pl.loop(0, nch, step=_NBUF)
        def _(t):
            for buf in range(_NBUF):
                c = t + buf
                wait_gather(buf)
                pltpu.sync_copy(rows_v.at[buf],
                                out_hbm.at[pl.ds(base + c * _CH, _CH)])

                @pl.when(c + _NBUF < nch)
                def _():
                    start_gather(buf, c + _NBUF)

    return k(idx, table)


def kernel(x, table):
    shp = x.shape
    out = _embed_gather(x.reshape(-1), table)
    return out.reshape(*shp, table.shape[1])
